# baseline (speedup 1.0000x reference)
import jax
import jax.numpy as jnp
import numpy as np
from jax.experimental import pallas as pl
from jax.experimental.pallas import tpu as pltpu

_KS = 3
_PAD = 1
_EPS = 1e-8

_C = 512
_H = 64
_W = 64
_HW = _H * _W
_CK = _C * _KS * _KS
_QT = 512
_PT = 512
_NQ = _HW // _QT
_NP = _HW // _PT
_NS_ITERS = 16


def _whiten(x):
    B, C, H, W = x.shape
    f = x.reshape(B, C, H * W)
    f = f - f.mean(-1, keepdims=True)
    cov = jnp.einsum('bcn,bdn->bcd', f, f) / (H * W - 1)
    e, v = jnp.linalg.eigh(cov)
    d = 1.0 / jnp.sqrt(jnp.maximum(e, _EPS))
    wmat = jnp.einsum('bce,be,bde->bcd', v, d, v)
    return jnp.einsum('bcd,bdn->bcn', wmat, f).reshape(B, C, H, W)


def _shifted_stack(x, axis):
    B, C, H, W = x.shape
    xp = jnp.pad(x, ((0, 0), (0, 0), (_PAD, _PAD), (_PAD, _PAD)))
    slabs = [xp[:, :, i:i + H, j:j + W].reshape(B, C, H * W)
             for i in range(_KS) for j in range(_KS)]
    return jnp.stack(slabs, axis=axis)


def _ns_body(cov_ref, cmat_ref):
    a = cov_ref[0]
    s2 = jnp.sum(a * a)
    rs = jax.lax.rsqrt(s2)
    ii = jax.lax.broadcasted_iota(jnp.int32, (_C, _C), 0)
    jj = jax.lax.broadcasted_iota(jnp.int32, (_C, _C), 1)
    eye1 = jnp.where(ii == jj, 1.0, 0.0)
    eye3 = jnp.where(ii == jj, 3.0, 0.0)

    def it(_, yz):
        y, z = yz
        t = eye3 - jnp.dot(z, y, preferred_element_type=jnp.float32)
        return (0.5 * jnp.dot(y, t, preferred_element_type=jnp.float32),
                0.5 * jnp.dot(t, z, preferred_element_type=jnp.float32))

    y, _ = jax.lax.fori_loop(0, _NS_ITERS, it, (a * rs, eye1))
    cmat_ref[0] = y * jnp.sqrt(jnp.sqrt(s2))


def _score_body(kn_ref, ct_ref, idx_ref, rmax_ref, ridx_ref):
    pi = pl.program_id(2)
    s = jax.lax.dot_general(kn_ref[0], ct_ref[0], (((1,), (0,)), ((), ())),
                            preferred_element_type=jnp.float32)
    tmax = jnp.max(s, axis=0, keepdims=True)
    ii = jax.lax.broadcasted_iota(jnp.int32, s.shape, 0)
    tidx = jnp.min(jnp.where(s == tmax, ii, _HW), axis=0,
                   keepdims=True) + pi * _PT

    @pl.when(pi == 0)
    def _init():
        rmax_ref[...] = tmax
        ridx_ref[...] = tidx

    @pl.when(pi != 0)
    def _update():
        better = tmax > rmax_ref[...]
        rmax_ref[...] = jnp.where(better, tmax, rmax_ref[...])
        ridx_ref[...] = jnp.where(better, tidx, ridx_ref[...])

    @pl.when(pi == _NP - 1)
    def _emit():
        idx_ref[0, 0] = ridx_ref[...]


def _recon_body(idxp_ref, ks_ref, cmat_ref, out_ref, acc_ref):
    yi = pl.program_id(1)
    pi = pl.program_id(2)
    win = idxp_ref[0, :, pl.ds(pl.multiple_of(yi * _QT, 128), 768)]

    lane = jax.lax.broadcasted_iota(jnp.int32, (1, _QT), 1)
    x = lane & 63
    yg = (lane >> 6) + yi * 8
    pio = jax.lax.broadcasted_iota(jnp.int32, (_PT, _QT), 0) + pi * _PT

    tot = jnp.zeros((_C, _QT), jnp.float32)
    for dx in range(_KS):
        for dy in range(_KS):
            sh = (3 - dx) * 64 + (1 - dy)
            idxs = win[:, sh:sh + _QT]
            yv = (yg >= dx - 1) & (yg <= 62 + dx)
            xv = (x >= dy - 1) & (x <= 62 + dy)
            oh = jnp.where((idxs == pio) & yv & xv, 1.0, 0.0)
            oh = oh.astype(jnp.bfloat16)
            tot = tot + jax.lax.dot_general(
                ks_ref[0, dx * 3 + dy], oh, (((1,), (0,)), ((), ())),
                preferred_element_type=jnp.float32)

    @pl.when(pi == 0)
    def _init():
        acc_ref[...] = tot

    @pl.when(pi != 0)
    def _add():
        acc_ref[...] = acc_ref[...] + tot

    @pl.when(pi == _NP - 1)
    def _emit():
        cy = jnp.where((yg == 0) | (yg == 63), 2.0, 3.0)
        cx = jnp.where((x == 0) | (x == 63), 2.0, 3.0)
        reassembled = acc_ref[...] / (cy * cx)
        out_ref[0] = jnp.dot(cmat_ref[0], reassembled,
                             preferred_element_type=jnp.float32)


def _patch_swap(nc, ns, cmat):
    B = nc.shape[0]

    kern = _shifted_stack(ns, 2).reshape(B, _C, _KS * _KS, _HW)
    kern = kern.transpose(0, 3, 1, 2).reshape(B, _HW, _CK)
    knorm = jnp.linalg.norm(kern, axis=2, keepdims=True) + 1e-5
    kn = (kern / knorm).astype(jnp.bfloat16)

    ct = _shifted_stack(nc, 2).reshape(B, _CK, _HW).astype(jnp.bfloat16)

    idx4 = pl.pallas_call(
        _score_body,
        grid=(B, _NQ, _NP),
        in_specs=[
            pl.BlockSpec((1, _PT, _CK), lambda b, q, p: (b, p, 0)),
            pl.BlockSpec((1, _CK, _QT), lambda b, q, p: (b, 0, q)),
        ],
        out_specs=pl.BlockSpec((1, 1, 1, _QT), lambda b, q, p: (b, q, 0, 0)),
        out_shape=jax.ShapeDtypeStruct((B, _NQ, 1, _QT), jnp.int32),
        scratch_shapes=[pltpu.VMEM((1, _QT), jnp.float32),
                        pltpu.VMEM((1, _QT), jnp.int32)],
        compiler_params=pltpu.CompilerParams(
            dimension_semantics=("parallel", "arbitrary", "arbitrary")),
    )(kn, ct)

    idx = idx4.reshape(B, _H, _W)
    idxp = jnp.pad(idx, ((0, 0), (2, 2), (0, 0))).reshape(B, 1, 68 * 64)

    ks = _shifted_stack(ns, 1).astype(jnp.bfloat16)

    out_flat = pl.pallas_call(
        _recon_body,
        grid=(B, _NQ, _NP),
        in_specs=[
            pl.BlockSpec((1, 1, 68 * 64), lambda b, y, p: (b, 0, 0)),
            pl.BlockSpec((1, 9, _C, _PT), lambda b, y, p: (b, 0, 0, p)),
            pl.BlockSpec((1, _C, _C), lambda b, y, p: (b, 0, 0)),
        ],
        out_specs=pl.BlockSpec((1, _C, _QT), lambda b, y, p: (b, 0, y)),
        out_shape=jax.ShapeDtypeStruct((B, _C, _HW), jnp.float32),
        scratch_shapes=[pltpu.VMEM((_C, _QT), jnp.float32)],
        compiler_params=pltpu.CompilerParams(
            dimension_semantics=("parallel", "arbitrary", "arbitrary")),
    )(idxp, ks, cmat)

    return out_flat


def kernel(content_feature, style_feature):
    nc = _whiten(content_feature)
    ns = _whiten(style_feature)
    B = nc.shape[0]

    t = style_feature.reshape(B, _C, -1)
    tm = t.mean(-1, keepdims=True)
    tc = t - tm
    cov = jnp.einsum('bcn,bdn->bcd', tc, tc) / (tc.shape[-1] - 1)
    cmat = pl.pallas_call(
        _ns_body,
        grid=(B,),
        in_specs=[pl.BlockSpec((1, _C, _C), lambda b: (b, 0, 0))],
        out_specs=pl.BlockSpec((1, _C, _C), lambda b: (b, 0, 0)),
        out_shape=jax.ShapeDtypeStruct((B, _C, _C), jnp.float32),
        compiler_params=pltpu.CompilerParams(
            dimension_semantics=("parallel",)),
    )(cov)

    out_flat = _patch_swap(nc, ns, cmat)
    return (out_flat + tm).reshape(B, _C, _H, _W)


# speedup vs baseline: 1.0008x; 1.0008x over previous
import jax
import jax.numpy as jnp
import numpy as np
from jax.experimental import pallas as pl
from jax.experimental.pallas import tpu as pltpu

_KS = 3
_PAD = 1
_EPS = 1e-8

_C = 512
_H = 64
_W = 64
_HW = _H * _W
_CK = _C * _KS * _KS
_QT = 512
_PT = 512
_NQ = _HW // _QT
_NP = _HW // _PT
_NS_ITERS = 16


def _whiten(x):
    B, C, H, W = x.shape
    f = x.reshape(B, C, H * W)
    f = f - f.mean(-1, keepdims=True)
    cov = jnp.einsum('bcn,bdn->bcd', f, f) / (H * W - 1)
    e, v = jnp.linalg.eigh(cov)
    d = 1.0 / jnp.sqrt(jnp.maximum(e, _EPS))
    wmat = jnp.einsum('bce,be,bde->bcd', v, d, v)
    return jnp.einsum('bcd,bdn->bcn', wmat, f).reshape(B, C, H, W)


def _shifted_stack(x, axis):
    B, C, H, W = x.shape
    xp = jnp.pad(x, ((0, 0), (0, 0), (_PAD, _PAD), (_PAD, _PAD)))
    slabs = [xp[:, :, i:i + H, j:j + W].reshape(B, C, H * W)
             for i in range(_KS) for j in range(_KS)]
    return jnp.stack(slabs, axis=axis)


def _ns_body(cov_ref, cmat_ref):
    a = cov_ref[0]
    s2 = jnp.sum(a * a)
    rs = jax.lax.rsqrt(s2)
    ii = jax.lax.broadcasted_iota(jnp.int32, (_C, _C), 0)
    jj = jax.lax.broadcasted_iota(jnp.int32, (_C, _C), 1)
    eye1 = jnp.where(ii == jj, 1.0, 0.0)
    eye3 = jnp.where(ii == jj, 3.0, 0.0)

    def it(_, yz):
        y, z = yz
        t = eye3 - jnp.dot(z, y, preferred_element_type=jnp.float32)
        return (0.5 * jnp.dot(y, t, preferred_element_type=jnp.float32),
                0.5 * jnp.dot(t, z, preferred_element_type=jnp.float32))

    y, _ = jax.lax.fori_loop(0, _NS_ITERS, it, (a * rs, eye1))
    cmat_ref[0] = y * jnp.sqrt(jnp.sqrt(s2))


def _score_body(kn_ref, ct_ref, idx_ref, rmax_ref, ridx_ref):
    pi = pl.program_id(2)
    s = jax.lax.dot_general(kn_ref[0], ct_ref[0], (((1,), (0,)), ((), ())),
                            preferred_element_type=jnp.float32)
    tmax = jnp.max(s, axis=0, keepdims=True)
    ii = jax.lax.broadcasted_iota(jnp.int32, s.shape, 0)
    tidx = jnp.min(jnp.where(s == tmax, ii, _HW), axis=0,
                   keepdims=True) + pi * _PT

    @pl.when(pi == 0)
    def _init():
        rmax_ref[...] = tmax
        ridx_ref[...] = tidx

    @pl.when(pi != 0)
    def _update():
        better = tmax > rmax_ref[...]
        rmax_ref[...] = jnp.where(better, tmax, rmax_ref[...])
        ridx_ref[...] = jnp.where(better, tidx, ridx_ref[...])

    @pl.when(pi == _NP - 1)
    def _emit():
        idx_ref[0, 0] = ridx_ref[...]


def _recon_body(idxp_ref, ks_ref, cmat_ref, out_ref, acc_ref):
    yi = pl.program_id(1)
    pi = pl.program_id(2)
    win = idxp_ref[0, :, pl.ds(pl.multiple_of(yi * _QT, 128), 768)]

    lane = jax.lax.broadcasted_iota(jnp.int32, (1, _QT), 1)
    x = lane & 63
    yg = (lane >> 6) + yi * 8
    pio = jax.lax.broadcasted_iota(jnp.int32, (_PT, _QT), 0) + pi * _PT

    tot = jnp.zeros((_C, _QT), jnp.float32)
    for dx in range(_KS):
        for dy in range(_KS):
            sh = (3 - dx) * 64 + (1 - dy)
            idxs = win[:, sh:sh + _QT]
            yv = (yg >= dx - 1) & (yg <= 62 + dx)
            xv = (x >= dy - 1) & (x <= 62 + dy)
            oh = jnp.where((idxs == pio) & yv & xv, 1.0, 0.0)
            oh = oh.astype(jnp.bfloat16)
            tot = tot + jax.lax.dot_general(
                ks_ref[0, dx * 3 + dy], oh, (((1,), (0,)), ((), ())),
                preferred_element_type=jnp.float32)

    @pl.when(pi == 0)
    def _init():
        acc_ref[...] = tot

    @pl.when(pi != 0)
    def _add():
        acc_ref[...] = acc_ref[...] + tot

    @pl.when(pi == _NP - 1)
    def _emit():
        cy = jnp.where((yg == 0) | (yg == 63), 2.0, 3.0)
        cx = jnp.where((x == 0) | (x == 63), 2.0, 3.0)
        reassembled = acc_ref[...] / (cy * cx)
        out_ref[0] = jnp.dot(cmat_ref[0], reassembled,
                             preferred_element_type=jnp.float32)


def _patch_swap(nc, ns, cmat):
    B = nc.shape[0]

    kern = _shifted_stack(ns, 2).reshape(B, _C, _KS * _KS, _HW)
    kern = kern.transpose(0, 3, 1, 2).reshape(B, _HW, _CK)
    knorm = jnp.linalg.norm(kern, axis=2, keepdims=True) + 1e-5
    kn = (kern / knorm).astype(jnp.bfloat16)

    ct = _shifted_stack(nc, 2).reshape(B, _CK, _HW).astype(jnp.bfloat16)

    idx4 = pl.pallas_call(
        _score_body,
        grid=(B, _NQ, _NP),
        in_specs=[
            pl.BlockSpec((1, _PT, _CK), lambda b, q, p: (b, p, 0)),
            pl.BlockSpec((1, _CK, _QT), lambda b, q, p: (b, 0, q)),
        ],
        out_specs=pl.BlockSpec((1, 1, 1, _QT), lambda b, q, p: (b, q, 0, 0)),
        out_shape=jax.ShapeDtypeStruct((B, _NQ, 1, _QT), jnp.int32),
        scratch_shapes=[pltpu.VMEM((1, _QT), jnp.float32),
                        pltpu.VMEM((1, _QT), jnp.int32)],
        compiler_params=pltpu.CompilerParams(
            dimension_semantics=("parallel", "arbitrary", "arbitrary")),
    )(kn, ct)

    idx = idx4.reshape(B, _H, _W)
    idxp = jnp.pad(idx, ((0, 0), (2, 2), (0, 0))).reshape(B, 1, 68 * 64)

    ks = _shifted_stack(ns, 1).astype(jnp.bfloat16)

    out_flat = pl.pallas_call(
        _recon_body,
        grid=(B, _NQ, _NP),
        in_specs=[
            pl.BlockSpec((1, 1, 68 * 64), lambda b, y, p: (b, 0, 0)),
            pl.BlockSpec((1, 9, _C, _PT), lambda b, y, p: (b, 0, 0, p)),
            pl.BlockSpec((1, _C, _C), lambda b, y, p: (b, 0, 0)),
        ],
        out_specs=pl.BlockSpec((1, _C, _QT), lambda b, y, p: (b, 0, y)),
        out_shape=jax.ShapeDtypeStruct((B, _C, _HW), jnp.float32),
        scratch_shapes=[pltpu.VMEM((_C, _QT), jnp.float32)],
        compiler_params=pltpu.CompilerParams(
            dimension_semantics=("parallel", "arbitrary", "arbitrary")),
    )(idxp, ks, cmat)

    return out_flat


def kernel(content_feature, style_feature):
    n = content_feature.shape[0]
    w = _whiten(jnp.concatenate([content_feature, style_feature], axis=0))
    nc, ns = w[:n], w[n:]
    B = nc.shape[0]

    t = style_feature.reshape(B, _C, -1)
    tm = t.mean(-1, keepdims=True)
    tc = t - tm
    cov = jnp.einsum('bcn,bdn->bcd', tc, tc) / (tc.shape[-1] - 1)
    cmat = pl.pallas_call(
        _ns_body,
        grid=(B,),
        in_specs=[pl.BlockSpec((1, _C, _C), lambda b: (b, 0, 0))],
        out_specs=pl.BlockSpec((1, _C, _C), lambda b: (b, 0, 0)),
        out_shape=jax.ShapeDtypeStruct((B, _C, _C), jnp.float32),
        compiler_params=pltpu.CompilerParams(
            dimension_semantics=("parallel",)),
    )(cov)

    out_flat = _patch_swap(nc, ns, cmat)
    return (out_flat + tm).reshape(B, _C, _H, _W)
